# baseline (speedup 1.0000x reference)
import functools

import jax
import jax.numpy as jnp
from jax import lax
from jax.experimental import pallas as pl
from jax.experimental.pallas import tpu as pltpu
from jax.experimental.pallas import tpu_sc as plsc

_IDX_CHUNK = 128


def _sc_gather(idx3d, emb_table):
    num_workers, chunks_per_w, chunk = idx3d.shape
    _, emb = emb_table.shape

    mesh = plsc.VectorSubcoreMesh(core_axis_name="c", subcore_axis_name="s")

    @functools.partial(
        pl.kernel,
        out_type=jax.ShapeDtypeStruct((num_workers * chunks_per_w, chunk, emb), jnp.float32),
        mesh=mesh,
        scratch_types=[
            pltpu.VMEM((chunks_per_w, chunk), jnp.int32),
            pltpu.VMEM((chunks_per_w, chunk, emb), jnp.float32),
            pltpu.SemaphoreType.DMA,
        ],
        compiler_params=pltpu.CompilerParams(use_tc_tiling_on_sc=False),
    )
    def gather_kernel(idx_hbm, table_hbm, out_hbm, idx_v, rows_v, sem):
        num_cores = jax.lax.axis_size("c")
        wid = lax.axis_index("s") * num_cores + lax.axis_index("c")
        pltpu.sync_copy(idx_hbm.at[wid], idx_v)
        copies = [
            pltpu.async_copy(table_hbm.at[idx_v.at[j]], rows_v.at[j], sem)
            for j in range(chunks_per_w)
        ]
        for c in copies:
            c.wait()
        pltpu.sync_copy(rows_v, out_hbm.at[pl.ds(wid * chunks_per_w, chunks_per_w)])

    return gather_kernel(idx3d, emb_table)


def _projection(flat, dense_w, dense_b2d, block_n):
    m, k = flat.shape
    n = dense_w.shape[1]
    grid = (pl.cdiv(n, block_n),)

    def mm_kernel(flat_ref, w_ref, b_ref, out_ref):
        out_ref[...] = (
            jnp.dot(flat_ref[...], w_ref[...], preferred_element_type=jnp.float32)
            + b_ref[...]
        )

    return pl.pallas_call(
        mm_kernel,
        grid=grid,
        in_specs=[
            pl.BlockSpec((m, k), lambda i: (0, 0)),
            pl.BlockSpec((k, block_n), lambda i: (0, i)),
            pl.BlockSpec((1, block_n), lambda i: (0, i)),
        ],
        out_specs=pl.BlockSpec((m, block_n), lambda i: (0, i)),
        out_shape=jax.ShapeDtypeStruct((m, n), jnp.float32),
        compiler_params=pltpu.CompilerParams(
            dimension_semantics=("arbitrary",),
        ),
    )(flat, dense_w, dense_b2d)


def kernel(context, emb_table, dense_w, dense_b):
    batch, ctx_len = context.shape
    vocab, emb = emb_table.shape
    total = batch * ctx_len
    info = plsc.get_sparse_core_info()
    num_workers = info.num_cores * info.num_subcores
    idx3d = context.astype(jnp.int32).reshape(
        num_workers, total // (num_workers * _IDX_CHUNK), _IDX_CHUNK
    )
    rows = _sc_gather(idx3d, emb_table)
    flat = rows.reshape(batch, ctx_len * emb)
    logits = _projection(flat, dense_w, dense_b.reshape(1, vocab), block_n=1024)
    return logits


# speedup vs baseline: 1.0649x; 1.0649x over previous
import functools

import jax
import jax.numpy as jnp
from jax import lax
from jax.experimental import pallas as pl
from jax.experimental.pallas import tpu as pltpu
from jax.experimental.pallas import tpu_sc as plsc

_IDX_CHUNK = 128


def _sc_gather(idx3d, emb_table):
    num_workers, chunks_per_w, chunk = idx3d.shape
    _, emb = emb_table.shape

    mesh = plsc.VectorSubcoreMesh(core_axis_name="c", subcore_axis_name="s")

    @functools.partial(
        pl.kernel,
        out_type=jax.ShapeDtypeStruct((num_workers * chunks_per_w, chunk, emb), jnp.float32),
        mesh=mesh,
        scratch_types=[
            pltpu.VMEM((chunks_per_w, chunk), jnp.int32),
            pltpu.VMEM((chunks_per_w, chunk, emb), jnp.float32),
            pltpu.SemaphoreType.DMA,
        ],
        compiler_params=pltpu.CompilerParams(use_tc_tiling_on_sc=False),
    )
    def gather_kernel(idx_hbm, table_hbm, out_hbm, idx_v, rows_v, sem):
        num_cores = jax.lax.axis_size("c")
        wid = lax.axis_index("s") * num_cores + lax.axis_index("c")
        pltpu.sync_copy(idx_hbm.at[wid], idx_v)
        copies = [
            pltpu.async_copy(table_hbm.at[idx_v.at[j]], rows_v.at[j], sem)
            for j in range(chunks_per_w)
        ]
        for c in copies:
            c.wait()
        pltpu.sync_copy(rows_v, out_hbm.at[pl.ds(wid * chunks_per_w, chunks_per_w)])

    return gather_kernel(idx3d, emb_table)


def _projection(flat, dense_w, dense_b2d, block_n):
    m, k = flat.shape
    n = dense_w.shape[1]
    grid = (pl.cdiv(n, block_n),)

    def mm_kernel(flat_ref, w_ref, b_ref, out_ref):
        out_ref[...] = (
            jnp.dot(flat_ref[...], w_ref[...], preferred_element_type=jnp.float32)
            + b_ref[...]
        )

    return pl.pallas_call(
        mm_kernel,
        grid=grid,
        in_specs=[
            pl.BlockSpec((m, k), lambda i: (0, 0)),
            pl.BlockSpec((k, block_n), lambda i: (0, i)),
            pl.BlockSpec((1, block_n), lambda i: (0, i)),
        ],
        out_specs=pl.BlockSpec((m, block_n), lambda i: (0, i)),
        out_shape=jax.ShapeDtypeStruct((m, n), jnp.float32),
        compiler_params=pltpu.CompilerParams(
            dimension_semantics=("arbitrary",),
        ),
    )(flat, dense_w, dense_b2d)


def kernel(context, emb_table, dense_w, dense_b):
    batch, ctx_len = context.shape
    vocab, emb = emb_table.shape
    total = batch * ctx_len
    info = plsc.get_sparse_core_info()
    num_workers = info.num_cores * info.num_subcores
    idx3d = context.astype(jnp.int32).reshape(
        num_workers, total // (num_workers * _IDX_CHUNK), _IDX_CHUNK
    )
    rows = _sc_gather(idx3d, emb_table)
    flat = rows.reshape(batch, ctx_len * emb)
    logits = _projection(flat, dense_w, dense_b.reshape(1, vocab), block_n=4096)
    return logits
